# baseline (speedup 1.0000x reference)
import functools

import jax
import jax.numpy as jnp
from jax import lax
from jax.experimental import pallas as pl
from jax.experimental.pallas import tpu as pltpu
from jax.experimental.pallas import tpu_sc as plsc

N = 10000
D = 128
DQ = 32
NQ = D // DQ
E = 320000
NC = 2
NS = 16
NW = NC * NS
EPW = E // NW
K = 80
NCHUNK = EPW // K
OFF0 = 632
LASTR = N - 15 * OFF0
DEGW = 16
RB = 2000

_mesh = plsc.VectorSubcoreMesh(core_axis_name="c", subcore_axis_name="s")


_NB_DEG = 5

_deg_kwargs = dict(
    out_type=jax.ShapeDtypeStruct((NC, N, DEGW), jnp.float32),
    mesh=_mesh,
    scratch_types=[
        pltpu.VMEM((NCHUNK, K), jnp.int32),
        pltpu.VMEM((K, DEGW), jnp.float32),
        pltpu.VMEM((OFF0, DEGW), jnp.float32),
        pltpu.VMEM_SHARED((N, DEGW), jnp.float32),
        pltpu.SemaphoreType.DMA((_NB_DEG,)),
    ],
    compiler_params=pltpu.CompilerParams(use_tc_tiling_on_sc=False),
)


def _deg_body(dst_hbm, out_hbm, dst_v, ones_v, zbuf_v, deg_sh, ssem):
    c = lax.axis_index("c")
    s = lax.axis_index("s")
    wid = s * NC + c
    one = jnp.full((16,), 1.0, jnp.float32)
    zero = jnp.zeros((16,), jnp.float32)

    def fill(i, _):
        ones_v[i] = one
        return 0

    lax.fori_loop(0, K, fill, 0)

    def fillz(i, _):
        zbuf_v[i] = zero
        return 0

    lax.fori_loop(0, OFF0, fillz, 0)
    pltpu.sync_copy(dst_hbm.at[wid], dst_v)

    @pl.when(s < 15)
    def _():
        base = pl.multiple_of(s * OFF0, 8)
        pltpu.sync_copy(zbuf_v, deg_sh.at[pl.ds(base, OFF0)])

    @pl.when(s == 15)
    def _():
        pltpu.sync_copy(zbuf_v.at[pl.ds(0, LASTR)],
                        deg_sh.at[pl.ds(15 * OFF0, LASTR)])

    plsc.subcore_barrier()

    def group(g, _):
        for b in range(_NB_DEG):
            @pl.when(g > 0)
            def _():
                pltpu.make_async_copy(ones_v, deg_sh.at[pl.ds(0, K)],
                                      ssem.at[b]).wait()

            pltpu.async_copy(ones_v, deg_sh.at[dst_v.at[g * _NB_DEG + b]],
                             ssem.at[b], add=True)
        return 0

    lax.fori_loop(0, NCHUNK // _NB_DEG, group, 0)
    for b in range(_NB_DEG):
        pltpu.make_async_copy(ones_v, deg_sh.at[pl.ds(0, K)],
                              ssem.at[b]).wait()
    plsc.subcore_barrier()

    @pl.when(s < 15)
    def _():
        base = pl.multiple_of(s * OFF0, 8)
        pltpu.sync_copy(deg_sh.at[pl.ds(base, OFF0)],
                        out_hbm.at[c, pl.ds(base, OFF0)])

    @pl.when(s == 15)
    def _():
        pltpu.sync_copy(deg_sh.at[pl.ds(15 * OFF0, LASTR)],
                        out_hbm.at[c, pl.ds(15 * OFF0, LASTR)])


_deg_kernel = pl.kernel(_deg_body, **_deg_kwargs)


NBUF = 5
NGRP = NCHUNK // NBUF

_agg_kwargs = dict(
    out_type=jax.ShapeDtypeStruct((NC, NQ, N, DQ), jnp.float32),
    mesh=_mesh,
    scratch_types=[
        pltpu.VMEM((NCHUNK, K), jnp.int32),
        pltpu.VMEM((NCHUNK, K), jnp.int32),
        pltpu.VMEM((3 * NBUF, K, DQ), jnp.float32),
        pltpu.VMEM((OFF0, DQ), jnp.float32),
        pltpu.VMEM_SHARED((N, DQ), jnp.float32),
        pltpu.SemaphoreType.DMA((3 * NBUF,)),
        pltpu.SemaphoreType.DMA((3 * NBUF,)),
    ],
    compiler_params=pltpu.CompilerParams(use_tc_tiling_on_sc=False),
)


def _agg_body(u0, u1, u2, u3, src_hbm, dst_hbm, out_hbm,
              src_v, dst_v, rows_v, zbuf, agg_sh, gsem, ssem):
    c = lax.axis_index("c")
    s = lax.axis_index("s")
    wid = s * NC + c
    zero = jnp.zeros((16,), jnp.float32)

    def fillz(i, _):
        for j in range(DQ // 16):
            zbuf[i, pl.ds(j * 16, 16)] = zero
        return 0

    lax.fori_loop(0, OFF0, fillz, 0)
    pltpu.sync_copy(src_hbm.at[wid], src_v)
    pltpu.sync_copy(dst_hbm.at[wid], dst_v)

    for q, u_hbm in enumerate((u0, u1, u2, u3)):
        @pl.when(s < 15)
        def _():
            base = pl.multiple_of(s * OFF0, 8)
            pltpu.sync_copy(zbuf, agg_sh.at[pl.ds(base, OFF0)])

        @pl.when(s == 15)
        def _():
            pltpu.sync_copy(zbuf.at[pl.ds(0, LASTR)],
                            agg_sh.at[pl.ds(15 * OFF0, LASTR)])

        plsc.subcore_barrier()

        for b in range(3 * NBUF):
            pltpu.async_copy(u_hbm.at[src_v.at[b]], rows_v.at[b],
                             gsem.at[b])

        def group(g, _, u_hbm=u_hbm):
            sb = (g % 3) * NBUF
            for b in range(NBUF):
                pltpu.make_async_copy(u_hbm.at[pl.ds(0, K)],
                                      rows_v.at[sb + b],
                                      gsem.at[sb + b]).wait()
                pltpu.async_copy(rows_v.at[sb + b],
                                 agg_sh.at[dst_v.at[g * NBUF + b]],
                                 ssem.at[sb + b], add=True)
            for b in range(NBUF):
                pltpu.make_async_copy(rows_v.at[sb + b],
                                      agg_sh.at[pl.ds(0, K)],
                                      ssem.at[sb + b]).wait()

                @pl.when(g < NGRP - 3)
                def _(b=b, u_hbm=u_hbm):
                    pltpu.async_copy(
                        u_hbm.at[src_v.at[(g + 3) * NBUF + b]],
                        rows_v.at[sb + b], gsem.at[sb + b])

            return 0

        lax.fori_loop(0, NGRP, group, 0)
        plsc.subcore_barrier()

        @pl.when(s < 15)
        def _(q=q):
            base = pl.multiple_of(s * OFF0, 8)
            pltpu.sync_copy(agg_sh.at[pl.ds(base, OFF0)],
                            out_hbm.at[c, q, pl.ds(base, OFF0)])

        @pl.when(s == 15)
        def _(q=q):
            pltpu.sync_copy(agg_sh.at[pl.ds(15 * OFF0, LASTR)],
                            out_hbm.at[c, q, pl.ds(15 * OFF0, LASTR)])


_agg_kernel = pl.kernel(_agg_body, **_agg_kwargs)


def _tc_pre_body(x_ref, w_ref, degp_ref, dinv_ref, u0_ref, u1_ref, u2_ref,
                 u3_ref):
    deg = degp_ref[0, :, 0:1] + degp_ref[1, :, 0:1] + 1.0
    dinv = lax.rsqrt(deg)
    dinv_b = jnp.broadcast_to(dinv, (RB, D))
    dinv_ref[...] = dinv_b
    h = jnp.dot(x_ref[...], w_ref[...], preferred_element_type=jnp.float32)
    u = dinv_b * h
    for q, uq_ref in enumerate((u0_ref, u1_ref, u2_ref, u3_ref)):
        uq_ref[...] = u[:, q * DQ:(q + 1) * DQ]


_uq_shape = jax.ShapeDtypeStruct((N, DQ), jnp.float32)
_uq_spec = pl.BlockSpec((RB, DQ), lambda i: (i, 0))

_tc_pre = pl.pallas_call(
    _tc_pre_body,
    grid=(N // RB,),
    in_specs=[
        pl.BlockSpec((RB, D), lambda i: (i, 0)),
        pl.BlockSpec((D, D), lambda i: (0, 0)),
        pl.BlockSpec((NC, RB, DEGW), lambda i: (0, i, 0)),
    ],
    out_specs=(pl.BlockSpec((RB, D), lambda i: (i, 0)),
               _uq_spec, _uq_spec, _uq_spec, _uq_spec),
    out_shape=(jax.ShapeDtypeStruct((N, D), jnp.float32),
               _uq_shape, _uq_shape, _uq_shape, _uq_shape),
)


def _psum(p_ref):
    return jnp.concatenate(
        [p_ref[0, q] + p_ref[1, q] for q in range(NQ)], axis=-1)


def _tc_mid_body(p_ref, u0_ref, u1_ref, u2_ref, u3_ref, dinv_ref, b_ref,
                 w_ref, o0_ref, o1_ref, o2_ref, o3_ref):
    u = jnp.concatenate(
        [u0_ref[...], u1_ref[...], u2_ref[...], u3_ref[...]], axis=-1)
    dinv = dinv_ref[...]
    z = dinv * (_psum(p_ref) + u) + b_ref[...]
    act = jnp.where(z >= 0, z, 0.01 * z)
    un = dinv * jnp.dot(act, w_ref[...], preferred_element_type=jnp.float32)
    for q, oq_ref in enumerate((o0_ref, o1_ref, o2_ref, o3_ref)):
        oq_ref[...] = un[:, q * DQ:(q + 1) * DQ]


_tc_mid = pl.pallas_call(
    _tc_mid_body,
    grid=(N // RB,),
    in_specs=[
        pl.BlockSpec((NC, NQ, RB, DQ), lambda i: (0, 0, i, 0)),
        _uq_spec, _uq_spec, _uq_spec, _uq_spec,
        pl.BlockSpec((RB, D), lambda i: (i, 0)),
        pl.BlockSpec((1, D), lambda i: (0, 0)),
        pl.BlockSpec((D, D), lambda i: (0, 0)),
    ],
    out_specs=(_uq_spec, _uq_spec, _uq_spec, _uq_spec),
    out_shape=(_uq_shape, _uq_shape, _uq_shape, _uq_shape),
)


def _tc_fin_body(p_ref, u0_ref, u1_ref, u2_ref, u3_ref, dinv_ref, b_ref,
                 wg_ref, bg_ref, gene_ref, hid_ref):
    u = jnp.concatenate(
        [u0_ref[...], u1_ref[...], u2_ref[...], u3_ref[...]], axis=-1)
    hidden = dinv_ref[...] * (_psum(p_ref) + u) + b_ref[...]
    hid_ref[...] = hidden
    g = jnp.dot(hidden, wg_ref[...], preferred_element_type=jnp.float32)
    gene_ref[...] = jnp.maximum(g + bg_ref[...], 0.0)


_tc_fin = pl.pallas_call(
    _tc_fin_body,
    grid=(N // RB,),
    in_specs=[
        pl.BlockSpec((NC, NQ, RB, DQ), lambda i: (0, 0, i, 0)),
        _uq_spec, _uq_spec, _uq_spec, _uq_spec,
        pl.BlockSpec((RB, D), lambda i: (i, 0)),
        pl.BlockSpec((1, D), lambda i: (0, 0)),
        pl.BlockSpec((D, D), lambda i: (0, 0)),
        pl.BlockSpec((1, D), lambda i: (0, 0)),
    ],
    out_specs=(pl.BlockSpec((RB, D), lambda i: (i, 0)),
               pl.BlockSpec((RB, D), lambda i: (i, 0))),
    out_shape=(jax.ShapeDtypeStruct((N, D), jnp.float32),
               jax.ShapeDtypeStruct((N, D), jnp.float32)),
)


def kernel(x, edge_index, W1, b1, W2, b2, W3, b3, W4, b4, W5, b5, Wg, bg):
    src = edge_index[0].reshape(NW, NCHUNK, K)
    dst = edge_index[1].reshape(NW, NCHUNK, K)
    degp = _deg_kernel(dst)
    dinv_b, u0, u1, u2, u3 = _tc_pre(x, W1, degp)
    for W, b in ((W2, b1), (W3, b2), (W4, b3), (W5, b4)):
        p = _agg_kernel(u0, u1, u2, u3, src, dst)
        u0, u1, u2, u3 = _tc_mid(p, u0, u1, u2, u3, dinv_b,
                                 b.reshape(1, D), W)
    p = _agg_kernel(u0, u1, u2, u3, src, dst)
    gene, hidden = _tc_fin(p, u0, u1, u2, u3, dinv_b, b5.reshape(1, D), Wg,
                           bg.reshape(1, D))
    return (gene, hidden)


# speedup vs baseline: 1.0203x; 1.0203x over previous
import functools

import jax
import jax.numpy as jnp
from jax import lax
from jax.experimental import pallas as pl
from jax.experimental.pallas import tpu as pltpu
from jax.experimental.pallas import tpu_sc as plsc

N = 10000
D = 128
DQ = 32
NQ = D // DQ
E = 320000
NC = 2
NS = 16
NW = NC * NS
EPW = E // NW
K = 80
NCHUNK = EPW // K
OFF0 = 632
LASTR = N - 15 * OFF0
DEGW = 16
RB = 2000

_mesh = plsc.VectorSubcoreMesh(core_axis_name="c", subcore_axis_name="s")


_NB_DEG = 5

_deg_kwargs = dict(
    out_type=jax.ShapeDtypeStruct((NC, N, DEGW), jnp.float32),
    mesh=_mesh,
    scratch_types=[
        pltpu.VMEM((NCHUNK, K), jnp.int32),
        pltpu.VMEM((K, DEGW), jnp.float32),
        pltpu.VMEM((OFF0, DEGW), jnp.float32),
        pltpu.VMEM_SHARED((N, DEGW), jnp.float32),
        pltpu.SemaphoreType.DMA((_NB_DEG,)),
    ],
    compiler_params=pltpu.CompilerParams(use_tc_tiling_on_sc=False),
)


def _deg_body(dst_hbm, out_hbm, dst_v, ones_v, zbuf_v, deg_sh, ssem):
    c = lax.axis_index("c")
    s = lax.axis_index("s")
    wid = s * NC + c
    one = jnp.full((16,), 1.0, jnp.float32)
    zero = jnp.zeros((16,), jnp.float32)

    def fill(i, _):
        ones_v[i] = one
        return 0

    lax.fori_loop(0, K, fill, 0)

    def fillz(i, _):
        zbuf_v[i] = zero
        return 0

    lax.fori_loop(0, OFF0, fillz, 0)
    pltpu.sync_copy(dst_hbm.at[wid], dst_v)

    @pl.when(s < 15)
    def _():
        base = pl.multiple_of(s * OFF0, 8)
        pltpu.sync_copy(zbuf_v, deg_sh.at[pl.ds(base, OFF0)])

    @pl.when(s == 15)
    def _():
        pltpu.sync_copy(zbuf_v.at[pl.ds(0, LASTR)],
                        deg_sh.at[pl.ds(15 * OFF0, LASTR)])

    plsc.subcore_barrier()

    def group(g, _):
        for b in range(_NB_DEG):
            @pl.when(g > 0)
            def _():
                pltpu.make_async_copy(ones_v, deg_sh.at[pl.ds(0, K)],
                                      ssem.at[b]).wait()

            pltpu.async_copy(ones_v, deg_sh.at[dst_v.at[g * _NB_DEG + b]],
                             ssem.at[b], add=True)
        return 0

    lax.fori_loop(0, NCHUNK // _NB_DEG, group, 0)
    for b in range(_NB_DEG):
        pltpu.make_async_copy(ones_v, deg_sh.at[pl.ds(0, K)],
                              ssem.at[b]).wait()
    plsc.subcore_barrier()

    @pl.when(s < 15)
    def _():
        base = pl.multiple_of(s * OFF0, 8)
        pltpu.sync_copy(deg_sh.at[pl.ds(base, OFF0)],
                        out_hbm.at[c, pl.ds(base, OFF0)])

    @pl.when(s == 15)
    def _():
        pltpu.sync_copy(deg_sh.at[pl.ds(15 * OFF0, LASTR)],
                        out_hbm.at[c, pl.ds(15 * OFF0, LASTR)])


_deg_kernel = pl.kernel(_deg_body, **_deg_kwargs)


NBUF = 5
NGRP = NCHUNK // NBUF

_agg_kwargs = dict(
    out_type=jax.ShapeDtypeStruct((NC, NQ, N, DQ), jnp.float32),
    mesh=_mesh,
    scratch_types=[
        pltpu.VMEM((NCHUNK, K), jnp.int32),
        pltpu.VMEM((NCHUNK, K), jnp.int32),
        pltpu.VMEM((2 * NBUF, K, DQ), jnp.float32),
        pltpu.VMEM((OFF0, DQ), jnp.float32),
        pltpu.VMEM_SHARED((N, DQ), jnp.float32),
        pltpu.SemaphoreType.DMA((2 * NBUF,)),
        pltpu.SemaphoreType.DMA((2 * NBUF,)),
    ],
    compiler_params=pltpu.CompilerParams(use_tc_tiling_on_sc=False),
)


def _agg_body(u0, u1, u2, u3, src_hbm, dst_hbm, out_hbm,
              src_v, dst_v, rows_v, zbuf, agg_sh, gsem, ssem):
    c = lax.axis_index("c")
    s = lax.axis_index("s")
    wid = s * NC + c
    zero = jnp.zeros((16,), jnp.float32)

    def fillz(i, _):
        for j in range(DQ // 16):
            zbuf[i, pl.ds(j * 16, 16)] = zero
        return 0

    lax.fori_loop(0, OFF0, fillz, 0)
    pltpu.sync_copy(src_hbm.at[wid], src_v)
    pltpu.sync_copy(dst_hbm.at[wid], dst_v)

    for q, u_hbm in enumerate((u0, u1, u2, u3)):
        @pl.when(s < 15)
        def _():
            base = pl.multiple_of(s * OFF0, 8)
            pltpu.sync_copy(zbuf, agg_sh.at[pl.ds(base, OFF0)])

        @pl.when(s == 15)
        def _():
            pltpu.sync_copy(zbuf.at[pl.ds(0, LASTR)],
                            agg_sh.at[pl.ds(15 * OFF0, LASTR)])

        plsc.subcore_barrier()

        for b in range(2 * NBUF):
            pltpu.async_copy(u_hbm.at[src_v.at[b]], rows_v.at[b],
                             gsem.at[b])

        def group(g, _, u_hbm=u_hbm):
            sb = (g % 2) * NBUF
            for b in range(NBUF):
                pltpu.make_async_copy(u_hbm.at[pl.ds(0, K)],
                                      rows_v.at[sb + b],
                                      gsem.at[sb + b]).wait()
                pltpu.async_copy(rows_v.at[sb + b],
                                 agg_sh.at[dst_v.at[g * NBUF + b]],
                                 ssem.at[sb + b], add=True)
            for b in range(NBUF):
                pltpu.make_async_copy(rows_v.at[sb + b],
                                      agg_sh.at[pl.ds(0, K)],
                                      ssem.at[sb + b]).wait()

                @pl.when(g < NGRP - 2)
                def _(b=b, u_hbm=u_hbm):
                    pltpu.async_copy(
                        u_hbm.at[src_v.at[(g + 2) * NBUF + b]],
                        rows_v.at[sb + b], gsem.at[sb + b])

            return 0

        lax.fori_loop(0, NGRP, group, 0)
        plsc.subcore_barrier()

        @pl.when(s < 15)
        def _(q=q):
            base = pl.multiple_of(s * OFF0, 8)
            pltpu.sync_copy(agg_sh.at[pl.ds(base, OFF0)],
                            out_hbm.at[c, q, pl.ds(base, OFF0)])

        @pl.when(s == 15)
        def _(q=q):
            pltpu.sync_copy(agg_sh.at[pl.ds(15 * OFF0, LASTR)],
                            out_hbm.at[c, q, pl.ds(15 * OFF0, LASTR)])


_agg_kernel = pl.kernel(_agg_body, **_agg_kwargs)


def _tc_pre_body(x_ref, w_ref, degp_ref, dinv_ref, u0_ref, u1_ref, u2_ref,
                 u3_ref):
    deg = degp_ref[0, :, 0:1] + degp_ref[1, :, 0:1] + 1.0
    dinv = lax.rsqrt(deg)
    dinv_b = jnp.broadcast_to(dinv, (RB, D))
    dinv_ref[...] = dinv_b
    h = jnp.dot(x_ref[...], w_ref[...], preferred_element_type=jnp.float32)
    u = dinv_b * h
    for q, uq_ref in enumerate((u0_ref, u1_ref, u2_ref, u3_ref)):
        uq_ref[...] = u[:, q * DQ:(q + 1) * DQ]


_uq_shape = jax.ShapeDtypeStruct((N, DQ), jnp.float32)
_uq_spec = pl.BlockSpec((RB, DQ), lambda i: (i, 0))

_tc_pre = pl.pallas_call(
    _tc_pre_body,
    grid=(N // RB,),
    in_specs=[
        pl.BlockSpec((RB, D), lambda i: (i, 0)),
        pl.BlockSpec((D, D), lambda i: (0, 0)),
        pl.BlockSpec((NC, RB, DEGW), lambda i: (0, i, 0)),
    ],
    out_specs=(pl.BlockSpec((RB, D), lambda i: (i, 0)),
               _uq_spec, _uq_spec, _uq_spec, _uq_spec),
    out_shape=(jax.ShapeDtypeStruct((N, D), jnp.float32),
               _uq_shape, _uq_shape, _uq_shape, _uq_shape),
)


def _psum(p_ref):
    return jnp.concatenate(
        [p_ref[0, q] + p_ref[1, q] for q in range(NQ)], axis=-1)


def _tc_mid_body(p_ref, u0_ref, u1_ref, u2_ref, u3_ref, dinv_ref, b_ref,
                 w_ref, o0_ref, o1_ref, o2_ref, o3_ref):
    u = jnp.concatenate(
        [u0_ref[...], u1_ref[...], u2_ref[...], u3_ref[...]], axis=-1)
    dinv = dinv_ref[...]
    z = dinv * (_psum(p_ref) + u) + b_ref[...]
    act = jnp.where(z >= 0, z, 0.01 * z)
    un = dinv * jnp.dot(act, w_ref[...], preferred_element_type=jnp.float32)
    for q, oq_ref in enumerate((o0_ref, o1_ref, o2_ref, o3_ref)):
        oq_ref[...] = un[:, q * DQ:(q + 1) * DQ]


_tc_mid = pl.pallas_call(
    _tc_mid_body,
    grid=(N // RB,),
    in_specs=[
        pl.BlockSpec((NC, NQ, RB, DQ), lambda i: (0, 0, i, 0)),
        _uq_spec, _uq_spec, _uq_spec, _uq_spec,
        pl.BlockSpec((RB, D), lambda i: (i, 0)),
        pl.BlockSpec((1, D), lambda i: (0, 0)),
        pl.BlockSpec((D, D), lambda i: (0, 0)),
    ],
    out_specs=(_uq_spec, _uq_spec, _uq_spec, _uq_spec),
    out_shape=(_uq_shape, _uq_shape, _uq_shape, _uq_shape),
)


def _tc_fin_body(p_ref, u0_ref, u1_ref, u2_ref, u3_ref, dinv_ref, b_ref,
                 wg_ref, bg_ref, gene_ref, hid_ref):
    u = jnp.concatenate(
        [u0_ref[...], u1_ref[...], u2_ref[...], u3_ref[...]], axis=-1)
    hidden = dinv_ref[...] * (_psum(p_ref) + u) + b_ref[...]
    hid_ref[...] = hidden
    g = jnp.dot(hidden, wg_ref[...], preferred_element_type=jnp.float32)
    gene_ref[...] = jnp.maximum(g + bg_ref[...], 0.0)


_tc_fin = pl.pallas_call(
    _tc_fin_body,
    grid=(N // RB,),
    in_specs=[
        pl.BlockSpec((NC, NQ, RB, DQ), lambda i: (0, 0, i, 0)),
        _uq_spec, _uq_spec, _uq_spec, _uq_spec,
        pl.BlockSpec((RB, D), lambda i: (i, 0)),
        pl.BlockSpec((1, D), lambda i: (0, 0)),
        pl.BlockSpec((D, D), lambda i: (0, 0)),
        pl.BlockSpec((1, D), lambda i: (0, 0)),
    ],
    out_specs=(pl.BlockSpec((RB, D), lambda i: (i, 0)),
               pl.BlockSpec((RB, D), lambda i: (i, 0))),
    out_shape=(jax.ShapeDtypeStruct((N, D), jnp.float32),
               jax.ShapeDtypeStruct((N, D), jnp.float32)),
)


def kernel(x, edge_index, W1, b1, W2, b2, W3, b3, W4, b4, W5, b5, Wg, bg):
    src = edge_index[0].reshape(NW, NCHUNK, K)
    dst = edge_index[1].reshape(NW, NCHUNK, K)
    degp = _deg_kernel(dst)
    dinv_b, u0, u1, u2, u3 = _tc_pre(x, W1, degp)
    for W, b in ((W2, b1), (W3, b2), (W4, b3), (W5, b4)):
        p = _agg_kernel(u0, u1, u2, u3, src, dst)
        u0, u1, u2, u3 = _tc_mid(p, u0, u1, u2, u3, dinv_b,
                                 b.reshape(1, D), W)
    p = _agg_kernel(u0, u1, u2, u3, src, dst)
    gene, hidden = _tc_fin(p, u0, u1, u2, u3, dinv_b, b5.reshape(1, D), Wg,
                           bg.reshape(1, D))
    return (gene, hidden)


# speedup vs baseline: 1.0283x; 1.0078x over previous
import functools

import jax
import jax.numpy as jnp
from jax import lax
from jax.experimental import pallas as pl
from jax.experimental.pallas import tpu as pltpu
from jax.experimental.pallas import tpu_sc as plsc

N = 10000
D = 128
DQ = 32
NQ = D // DQ
E = 320000
NC = 2
NS = 16
NW = NC * NS
EPW = E // NW
K = 100
NCHUNK = EPW // K
OFF0 = 632
LASTR = N - 15 * OFF0
DEGW = 16
RB = 2000

_mesh = plsc.VectorSubcoreMesh(core_axis_name="c", subcore_axis_name="s")


_NB_DEG = 5

_deg_kwargs = dict(
    out_type=jax.ShapeDtypeStruct((NC, N, DEGW), jnp.float32),
    mesh=_mesh,
    scratch_types=[
        pltpu.VMEM((NCHUNK, K), jnp.int32),
        pltpu.VMEM((K, DEGW), jnp.float32),
        pltpu.VMEM((OFF0, DEGW), jnp.float32),
        pltpu.VMEM_SHARED((N, DEGW), jnp.float32),
        pltpu.SemaphoreType.DMA((_NB_DEG,)),
    ],
    compiler_params=pltpu.CompilerParams(use_tc_tiling_on_sc=False),
)


def _deg_body(dst_hbm, out_hbm, dst_v, ones_v, zbuf_v, deg_sh, ssem):
    c = lax.axis_index("c")
    s = lax.axis_index("s")
    wid = s * NC + c
    one = jnp.full((16,), 1.0, jnp.float32)
    zero = jnp.zeros((16,), jnp.float32)

    def fill(i, _):
        ones_v[i] = one
        return 0

    lax.fori_loop(0, K, fill, 0)

    def fillz(i, _):
        zbuf_v[i] = zero
        return 0

    lax.fori_loop(0, OFF0, fillz, 0)
    pltpu.sync_copy(dst_hbm.at[wid], dst_v)

    @pl.when(s < 15)
    def _():
        base = pl.multiple_of(s * OFF0, 8)
        pltpu.sync_copy(zbuf_v, deg_sh.at[pl.ds(base, OFF0)])

    @pl.when(s == 15)
    def _():
        pltpu.sync_copy(zbuf_v.at[pl.ds(0, LASTR)],
                        deg_sh.at[pl.ds(15 * OFF0, LASTR)])

    plsc.subcore_barrier()

    def group(g, _):
        for b in range(_NB_DEG):
            @pl.when(g > 0)
            def _():
                pltpu.make_async_copy(ones_v, deg_sh.at[pl.ds(0, K)],
                                      ssem.at[b]).wait()

            pltpu.async_copy(ones_v, deg_sh.at[dst_v.at[g * _NB_DEG + b]],
                             ssem.at[b], add=True)
        return 0

    lax.fori_loop(0, NCHUNK // _NB_DEG, group, 0)
    for b in range(_NB_DEG):
        pltpu.make_async_copy(ones_v, deg_sh.at[pl.ds(0, K)],
                              ssem.at[b]).wait()
    plsc.subcore_barrier()

    @pl.when(s < 15)
    def _():
        base = pl.multiple_of(s * OFF0, 8)
        pltpu.sync_copy(deg_sh.at[pl.ds(base, OFF0)],
                        out_hbm.at[c, pl.ds(base, OFF0)])

    @pl.when(s == 15)
    def _():
        pltpu.sync_copy(deg_sh.at[pl.ds(15 * OFF0, LASTR)],
                        out_hbm.at[c, pl.ds(15 * OFF0, LASTR)])


_deg_kernel = pl.kernel(_deg_body, **_deg_kwargs)


NBUF = 5
NGRP = NCHUNK // NBUF

_agg_kwargs = dict(
    out_type=jax.ShapeDtypeStruct((NC, NQ, N, DQ), jnp.float32),
    mesh=_mesh,
    scratch_types=[
        pltpu.VMEM((NCHUNK, K), jnp.int32),
        pltpu.VMEM((NCHUNK, K), jnp.int32),
        pltpu.VMEM((2 * NBUF, K, DQ), jnp.float32),
        pltpu.VMEM((OFF0, DQ), jnp.float32),
        pltpu.VMEM_SHARED((N, DQ), jnp.float32),
        pltpu.SemaphoreType.DMA((2 * NBUF,)),
        pltpu.SemaphoreType.DMA((2 * NBUF,)),
    ],
    compiler_params=pltpu.CompilerParams(use_tc_tiling_on_sc=False),
)


def _agg_body(u0, u1, u2, u3, src_hbm, dst_hbm, out_hbm,
              src_v, dst_v, rows_v, zbuf, agg_sh, gsem, ssem):
    c = lax.axis_index("c")
    s = lax.axis_index("s")
    wid = s * NC + c
    zero = jnp.zeros((16,), jnp.float32)

    def fillz(i, _):
        for j in range(DQ // 16):
            zbuf[i, pl.ds(j * 16, 16)] = zero
        return 0

    lax.fori_loop(0, OFF0, fillz, 0)
    pltpu.sync_copy(src_hbm.at[wid], src_v)
    pltpu.sync_copy(dst_hbm.at[wid], dst_v)

    for q, u_hbm in enumerate((u0, u1, u2, u3)):
        @pl.when(s < 15)
        def _():
            base = pl.multiple_of(s * OFF0, 8)
            pltpu.sync_copy(zbuf, agg_sh.at[pl.ds(base, OFF0)])

        @pl.when(s == 15)
        def _():
            pltpu.sync_copy(zbuf.at[pl.ds(0, LASTR)],
                            agg_sh.at[pl.ds(15 * OFF0, LASTR)])

        plsc.subcore_barrier()

        for b in range(2 * NBUF):
            pltpu.async_copy(u_hbm.at[src_v.at[b]], rows_v.at[b],
                             gsem.at[b])

        def group(g, _, u_hbm=u_hbm):
            sb = (g % 2) * NBUF
            for b in range(NBUF):
                pltpu.make_async_copy(u_hbm.at[pl.ds(0, K)],
                                      rows_v.at[sb + b],
                                      gsem.at[sb + b]).wait()
                pltpu.async_copy(rows_v.at[sb + b],
                                 agg_sh.at[dst_v.at[g * NBUF + b]],
                                 ssem.at[sb + b], add=True)
            for b in range(NBUF):
                pltpu.make_async_copy(rows_v.at[sb + b],
                                      agg_sh.at[pl.ds(0, K)],
                                      ssem.at[sb + b]).wait()

                @pl.when(g < NGRP - 2)
                def _(b=b, u_hbm=u_hbm):
                    pltpu.async_copy(
                        u_hbm.at[src_v.at[(g + 2) * NBUF + b]],
                        rows_v.at[sb + b], gsem.at[sb + b])

            return 0

        lax.fori_loop(0, NGRP, group, 0)
        plsc.subcore_barrier()

        @pl.when(s < 15)
        def _(q=q):
            base = pl.multiple_of(s * OFF0, 8)
            pltpu.sync_copy(agg_sh.at[pl.ds(base, OFF0)],
                            out_hbm.at[c, q, pl.ds(base, OFF0)])

        @pl.when(s == 15)
        def _(q=q):
            pltpu.sync_copy(agg_sh.at[pl.ds(15 * OFF0, LASTR)],
                            out_hbm.at[c, q, pl.ds(15 * OFF0, LASTR)])


_agg_kernel = pl.kernel(_agg_body, **_agg_kwargs)


def _tc_pre_body(x_ref, w_ref, degp_ref, dinv_ref, u0_ref, u1_ref, u2_ref,
                 u3_ref):
    deg = degp_ref[0, :, 0:1] + degp_ref[1, :, 0:1] + 1.0
    dinv = lax.rsqrt(deg)
    dinv_b = jnp.broadcast_to(dinv, (RB, D))
    dinv_ref[...] = dinv_b
    h = jnp.dot(x_ref[...], w_ref[...], preferred_element_type=jnp.float32)
    u = dinv_b * h
    for q, uq_ref in enumerate((u0_ref, u1_ref, u2_ref, u3_ref)):
        uq_ref[...] = u[:, q * DQ:(q + 1) * DQ]


_uq_shape = jax.ShapeDtypeStruct((N, DQ), jnp.float32)
_uq_spec = pl.BlockSpec((RB, DQ), lambda i: (i, 0))

_tc_pre = pl.pallas_call(
    _tc_pre_body,
    grid=(N // RB,),
    in_specs=[
        pl.BlockSpec((RB, D), lambda i: (i, 0)),
        pl.BlockSpec((D, D), lambda i: (0, 0)),
        pl.BlockSpec((NC, RB, DEGW), lambda i: (0, i, 0)),
    ],
    out_specs=(pl.BlockSpec((RB, D), lambda i: (i, 0)),
               _uq_spec, _uq_spec, _uq_spec, _uq_spec),
    out_shape=(jax.ShapeDtypeStruct((N, D), jnp.float32),
               _uq_shape, _uq_shape, _uq_shape, _uq_shape),
)


def _psum(p_ref):
    return jnp.concatenate(
        [p_ref[0, q] + p_ref[1, q] for q in range(NQ)], axis=-1)


def _tc_mid_body(p_ref, u0_ref, u1_ref, u2_ref, u3_ref, dinv_ref, b_ref,
                 w_ref, o0_ref, o1_ref, o2_ref, o3_ref):
    u = jnp.concatenate(
        [u0_ref[...], u1_ref[...], u2_ref[...], u3_ref[...]], axis=-1)
    dinv = dinv_ref[...]
    z = dinv * (_psum(p_ref) + u) + b_ref[...]
    act = jnp.where(z >= 0, z, 0.01 * z)
    un = dinv * jnp.dot(act, w_ref[...], preferred_element_type=jnp.float32)
    for q, oq_ref in enumerate((o0_ref, o1_ref, o2_ref, o3_ref)):
        oq_ref[...] = un[:, q * DQ:(q + 1) * DQ]


_tc_mid = pl.pallas_call(
    _tc_mid_body,
    grid=(N // RB,),
    in_specs=[
        pl.BlockSpec((NC, NQ, RB, DQ), lambda i: (0, 0, i, 0)),
        _uq_spec, _uq_spec, _uq_spec, _uq_spec,
        pl.BlockSpec((RB, D), lambda i: (i, 0)),
        pl.BlockSpec((1, D), lambda i: (0, 0)),
        pl.BlockSpec((D, D), lambda i: (0, 0)),
    ],
    out_specs=(_uq_spec, _uq_spec, _uq_spec, _uq_spec),
    out_shape=(_uq_shape, _uq_shape, _uq_shape, _uq_shape),
)


def _tc_fin_body(p_ref, u0_ref, u1_ref, u2_ref, u3_ref, dinv_ref, b_ref,
                 wg_ref, bg_ref, gene_ref, hid_ref):
    u = jnp.concatenate(
        [u0_ref[...], u1_ref[...], u2_ref[...], u3_ref[...]], axis=-1)
    hidden = dinv_ref[...] * (_psum(p_ref) + u) + b_ref[...]
    hid_ref[...] = hidden
    g = jnp.dot(hidden, wg_ref[...], preferred_element_type=jnp.float32)
    gene_ref[...] = jnp.maximum(g + bg_ref[...], 0.0)


_tc_fin = pl.pallas_call(
    _tc_fin_body,
    grid=(N // RB,),
    in_specs=[
        pl.BlockSpec((NC, NQ, RB, DQ), lambda i: (0, 0, i, 0)),
        _uq_spec, _uq_spec, _uq_spec, _uq_spec,
        pl.BlockSpec((RB, D), lambda i: (i, 0)),
        pl.BlockSpec((1, D), lambda i: (0, 0)),
        pl.BlockSpec((D, D), lambda i: (0, 0)),
        pl.BlockSpec((1, D), lambda i: (0, 0)),
    ],
    out_specs=(pl.BlockSpec((RB, D), lambda i: (i, 0)),
               pl.BlockSpec((RB, D), lambda i: (i, 0))),
    out_shape=(jax.ShapeDtypeStruct((N, D), jnp.float32),
               jax.ShapeDtypeStruct((N, D), jnp.float32)),
)


def kernel(x, edge_index, W1, b1, W2, b2, W3, b3, W4, b4, W5, b5, Wg, bg):
    src = edge_index[0].reshape(NW, NCHUNK, K)
    dst = edge_index[1].reshape(NW, NCHUNK, K)
    degp = _deg_kernel(dst)
    dinv_b, u0, u1, u2, u3 = _tc_pre(x, W1, degp)
    for W, b in ((W2, b1), (W3, b2), (W4, b3), (W5, b4)):
        p = _agg_kernel(u0, u1, u2, u3, src, dst)
        u0, u1, u2, u3 = _tc_mid(p, u0, u1, u2, u3, dinv_b,
                                 b.reshape(1, D), W)
    p = _agg_kernel(u0, u1, u2, u3, src, dst)
    gene, hidden = _tc_fin(p, u0, u1, u2, u3, dinv_b, b5.reshape(1, D), Wg,
                           bg.reshape(1, D))
    return (gene, hidden)
